# Initial kernel scaffold; baseline (speedup 1.0000x reference)
import jax
import jax.numpy as jnp
from jax.experimental import pallas as pl


def kernel(x, edge_index, Wl0, Wr0, a0, b0, Wl1, Wr1, a1, b1, Wl2, Wr2, a2, b2, Wp1, bp1, Wp2, bp2, Ws1, bs1, Ws2, bs2):
    raise NotImplementedError("write your pallas kernel here")


# speedup vs baseline: 15.4467x; 15.4467x over previous
import functools

import numpy as _np

import jax
import jax.numpy as jnp
from jax import lax
from jax.experimental import pallas as pl
from jax.experimental.pallas import tpu as pltpu
import jax.experimental.pallas.tpu_sc as plsc

N = 10000
DIN = 128
HID = 16
HEADS = 8
HC = HID * HEADS
NPAD = 10240
NC = 2
NS = 16
NW = NC * NS
K = 64
BLK = 1024
GRID = NPAD // BLK

_f32 = jnp.float32


def _prep0_body(x_ref, wl_ref, wr_ref, sa_ref, xl_ref, xr_ref, pb_ref, qb_ref):
    h = x_ref[...]
    xl = jnp.dot(h, wl_ref[...], preferred_element_type=_f32)
    xl_ref[...] = xl
    pb_ref[...] = jnp.dot(jnp.abs(xl), sa_ref[...], preferred_element_type=_f32)
    xr = jnp.dot(h, wr_ref[...], preferred_element_type=_f32)
    xr_ref[...] = xr
    qb_ref[...] = jnp.dot(jnp.abs(xr), sa_ref[...], preferred_element_type=_f32)


def _prep_body(acc_ref, den_ref, b_ref, wl_ref, wr_ref, sa_ref, s8_ref,
               xl_ref, xr_ref, pb_ref, qb_ref):
    feat = acc_ref[0] + acc_ref[1]
    den = jnp.sum(den_ref[...], axis=0)
    denb = jnp.dot(den, s8_ref[...], preferred_element_type=_f32)
    h = feat / jnp.maximum(denb, 1e-30) + b_ref[...]
    h = jnp.where(h > 0, h, jnp.exp(h) - 1.0)
    xl = jnp.dot(h, wl_ref[...], preferred_element_type=_f32)
    xl_ref[...] = xl
    pb_ref[...] = jnp.dot(jnp.abs(xl), sa_ref[...], preferred_element_type=_f32)
    xr = jnp.dot(h, wr_ref[...], preferred_element_type=_f32)
    xr_ref[...] = xr
    qb_ref[...] = jnp.dot(jnp.abs(xr), sa_ref[...], preferred_element_type=_f32)


def _final_body(acc_ref, den_ref, s8_ref, m_ref, b2_ref, wp1_ref, bp1_ref,
                wp2_ref, bp2_ref, ws1_ref, bs1_ref, ws2_ref, bs2_ref,
                pam_ref, sur_ref):
    feat = acc_ref[0] + acc_ref[1]
    den = jnp.sum(den_ref[...], axis=0)
    denb = jnp.dot(den, s8_ref[...], preferred_element_type=_f32)
    hn = feat / jnp.maximum(denb, 1e-30)
    h3 = jnp.dot(hn, m_ref[...], preferred_element_type=_f32) + b2_ref[...]
    r1 = jnp.maximum(jnp.dot(h3, wp1_ref[...], preferred_element_type=_f32) + bp1_ref[...], 0.0)
    pam_ref[...] = jnp.dot(r1, wp2_ref[...], preferred_element_type=_f32) + bp2_ref[...]
    r2 = jnp.maximum(jnp.dot(h3, ws1_ref[...], preferred_element_type=_f32) + bs1_ref[...], 0.0)
    sur_ref[...] = jnp.dot(r2, ws2_ref[...], preferred_element_type=_f32) + bs2_ref[...]


def _row_spec(cols):
    return pl.BlockSpec((BLK, cols), lambda i: (i, 0))


def _full_spec(shape):
    return pl.BlockSpec(shape, lambda i: tuple(0 for _ in shape))


_ACC_SPEC = pl.BlockSpec((NC, BLK, HC), lambda i: (0, i, 0))
_DEN_SPEC = pl.BlockSpec((NC, BLK, HEADS), lambda i: (0, i, 0))
_PREP_OUT_SPECS = [_row_spec(HC), _row_spec(HC), _row_spec(HEADS), _row_spec(HEADS)]
_PREP_OUT_SHAPE = [jax.ShapeDtypeStruct((NPAD, HC), _f32),
                   jax.ShapeDtypeStruct((NPAD, HC), _f32),
                   jax.ShapeDtypeStruct((NPAD, HEADS), _f32),
                   jax.ShapeDtypeStruct((NPAD, HEADS), _f32)]


def _tc_prep0(xp, wl, wr, sa):
    return pl.pallas_call(
        _prep0_body,
        grid=(GRID,),
        in_specs=[_row_spec(DIN), _full_spec((DIN, HC)), _full_spec((DIN, HC)),
                  _full_spec((HC, HEADS))],
        out_specs=_PREP_OUT_SPECS,
        out_shape=_PREP_OUT_SHAPE,
    )(xp, wl, wr, sa)


def _tc_prep(acc, den, b2d, wl, wr, sa, s8):
    return pl.pallas_call(
        _prep_body,
        grid=(GRID,),
        in_specs=[_ACC_SPEC, _DEN_SPEC,
                  _full_spec((1, HC)), _full_spec((HC, HC)), _full_spec((HC, HC)),
                  _full_spec((HC, HEADS)), _full_spec((HEADS, HC))],
        out_specs=_PREP_OUT_SPECS,
        out_shape=_PREP_OUT_SHAPE,
    )(acc, den, b2d, wl, wr, sa, s8)


def _tc_final(acc, den, s8, m, b2, wp1, bp1, wp2, bp2, ws1, bs1, ws2, bs2):
    return pl.pallas_call(
        _final_body,
        grid=(GRID,),
        in_specs=[_ACC_SPEC, _DEN_SPEC,
                  _full_spec((HEADS, HC)), _full_spec((HC, HID)), _full_spec((1, HID)),
                  _full_spec((HID, HID // 2)), _full_spec((1, HID // 2)),
                  _full_spec((HID // 2, 4)), _full_spec((1, 4)),
                  _full_spec((HID, HID // 2)), _full_spec((1, HID // 2)),
                  _full_spec((HID // 2, 1)), _full_spec((1, 1))],
        out_specs=[_row_spec(4), _row_spec(1)],
        out_shape=[jax.ShapeDtypeStruct((NPAD, 4), _f32),
                   jax.ShapeDtypeStruct((NPAD, 1), _f32)],
    )(acc, den, s8, m, b2, wp1, bp1, wp2, bp2, ws1, bs1, ws2, bs2)


NG = NPAD // 16


def _sc_edge_body(nblk, xl_hbm, xr_hbm, a_hbm, cg_hbm, src_hbm, dst_hbm, z_hbm,
                  accf_hbm, den_hbm,
                  sidx, didx, didx16, xlg, xrg, orows, orows2, av, cgv,
                  sem, sem2, acc, accd):
    c = lax.axis_index("c")
    s = lax.axis_index("s")
    wid = s * NC + c
    rows_per_sub = NPAD // NS
    grows_per_sub = NG // NS

    pltpu.sync_copy(z_hbm, acc.at[pl.ds(s * rows_per_sub, rows_per_sub)])
    pltpu.sync_copy(z_hbm.at[pl.ds(0, grows_per_sub)],
                    accd.at[pl.ds(s * grows_per_sub, grows_per_sub)])
    pltpu.sync_copy(a_hbm, av)
    pltpu.sync_copy(cg_hbm, cgv)
    plsc.subcore_barrier()

    lanes = lax.iota(jnp.int32, 16)
    lanes_f = lanes.astype(_f32)
    onehots = []
    for h in range(HEADS):
        dh = lanes_f - float(h)
        onehots.append(jnp.maximum(1.0 - dh * dh, 0.0))

    def blk_body(b, carry):
        pltpu.sync_copy(src_hbm.at[wid, pl.ds(b * K, K)], sidx)
        pltpu.sync_copy(dst_hbm.at[wid, pl.ds(b * K, K)], didx)
        cp1 = pltpu.async_copy(xl_hbm.at[sidx], xlg, sem)
        cp2 = pltpu.async_copy(xr_hbm.at[didx], xrg, sem2)
        for j in range(K // 16):
            didx16[pl.ds(j * 16, 16)] = jnp.right_shift(didx[pl.ds(j * 16, 16)], 4)
        cp1.wait()
        cp2.wait()

        def edge_body(e, carry2):
            pacc = jnp.zeros((16,), _f32)
            for h in range(HEADS):
                xlv = xlg[e, pl.ds(h * HID, HID)]
                xrv = xrg[e, pl.ds(h * HID, HID)]
                v = xlv + xrv
                v = jnp.maximum(v, 0.2 * v)
                t = v * av[h]
                for k in (1, 2, 4, 8):
                    t = t + t.at[lanes ^ k].get(mode="promise_in_bounds")
                pv = jnp.exp(t - cgv[h])
                orows[e, pl.ds(h * HID, HID)] = pv * xlv
                pacc = pacc + pv * onehots[h]
            eb = (e // 16) * 16
            dchunk = didx[pl.ds(eb, 16)]
            dsplat = dchunk.at[jnp.full((16,), e - eb, jnp.int32)].get(
                mode="promise_in_bounds")
            s16 = dsplat & 15
            m = (s16 & 1).astype(_f32)
            rot8 = pacc.at[lanes ^ 8].get(mode="promise_in_bounds")
            rot = pacc + m * (rot8 - pacc)
            qf = (s16 >> 1).astype(_f32)
            for j in range(8):
                dj = qf - float(j)
                mj = jnp.maximum(1.0 - dj * dj, 0.0)
                orows2[e, pl.ds(j * 16, 16)] = rot * mj
            return carry2

        lax.fori_loop(0, K, edge_body, 0)
        pltpu.sync_copy(orows, acc.at[didx], add=True)
        pltpu.sync_copy(orows2, accd.at[didx16], add=True)
        return carry

    lax.fori_loop(0, nblk, blk_body, 0)
    plsc.subcore_barrier()
    pltpu.sync_copy(acc.at[pl.ds(s * rows_per_sub, rows_per_sub)],
                    accf_hbm.at[c, pl.ds(s * rows_per_sub, rows_per_sub)])
    pltpu.sync_copy(accd.at[pl.ds(s * grows_per_sub, grows_per_sub)],
                    den_hbm.at[c, pl.ds(s * grows_per_sub, grows_per_sub)])


def _sc_edge(xl, xr, a, cgb, srcm, dstm, zrows, nblk):
    mesh = plsc.VectorSubcoreMesh(core_axis_name="c", subcore_axis_name="s")
    f = pl.kernel(
        functools.partial(_sc_edge_body, nblk),
        out_type=(jax.ShapeDtypeStruct((NC, NPAD, HC), _f32),
                  jax.ShapeDtypeStruct((NC, NG, HC), _f32)),
        mesh=mesh,
        scratch_types=[
            pltpu.VMEM((K,), jnp.int32),
            pltpu.VMEM((K,), jnp.int32),
            pltpu.VMEM((K,), jnp.int32),
            pltpu.VMEM((K, HC), _f32),
            pltpu.VMEM((K, HC), _f32),
            pltpu.VMEM((K, HC), _f32),
            pltpu.VMEM((K, HC), _f32),
            pltpu.VMEM((HEADS, HID), _f32),
            pltpu.VMEM((HEADS, HID), _f32),
            pltpu.SemaphoreType.DMA,
            pltpu.SemaphoreType.DMA,
            pltpu.VMEM_SHARED((NPAD, HC), _f32),
            pltpu.VMEM_SHARED((NG, HC), _f32),
        ],
    )
    return f(xl, xr, a, cgb, srcm, dstm, zrows)


def _blockdiag_absa(a):
    return (jnp.abs(a)[:, :, None] * jnp.eye(HEADS, dtype=_f32)[:, None, :]).reshape(HC, HEADS)


def kernel(x, edge_index, Wl0, Wr0, a0, b0, Wl1, Wr1, a1, b1, Wl2, Wr2, a2, b2,
           Wp1, bp1, Wp2, bp2, Ws1, bs1, Ws2, bs2):
    n = x.shape[0]
    e = edge_index.shape[1]
    etot = e + n
    ept = ((etot + NW * K - 1) // (NW * K)) * K
    epad = ept * NW
    nblk = ept // K

    loop = jnp.arange(n, dtype=edge_index.dtype)
    fill_src = jnp.zeros((epad - etot,), edge_index.dtype)
    fill_dst = jnp.full((epad - etot,), NPAD - 2, edge_index.dtype)
    src = jnp.concatenate([edge_index[0], loop, fill_src]).reshape(NW, ept)
    dst = jnp.concatenate([edge_index[1], loop, fill_dst]).reshape(NW, ept)

    xp = jnp.pad(x, ((0, NPAD - n), (0, 0)))
    zrows = jnp.zeros((NPAD // NS, HC), _f32)
    s8 = jnp.kron(jnp.eye(HEADS, dtype=_f32), jnp.ones((1, HID), _f32))
    m = jnp.kron(jnp.ones((HEADS, 1), _f32), jnp.eye(HID, dtype=_f32)) / HEADS

    def cg_of(pb, qb):
        c = jnp.max(pb, axis=0) + jnp.max(qb, axis=0)
        return jnp.broadcast_to(c[:, None], (HEADS, HID)).astype(_f32)

    xl, xr, pb, qb = _tc_prep0(xp, Wl0, Wr0, _blockdiag_absa(a0))
    acc, den = _sc_edge(xl, xr, a0, cg_of(pb, qb), src, dst, zrows, nblk)
    den = den.reshape(NC, NPAD, HEADS)
    xl, xr, pb, qb = _tc_prep(acc, den, b0.reshape(1, HC), Wl1, Wr1,
                              _blockdiag_absa(a1), s8)
    acc, den = _sc_edge(xl, xr, a1, cg_of(pb, qb), src, dst, zrows, nblk)
    den = den.reshape(NC, NPAD, HEADS)
    xl, xr, pb, qb = _tc_prep(acc, den, b1.reshape(1, HC), Wl2, Wr2,
                              _blockdiag_absa(a2), s8)
    acc, den = _sc_edge(xl, xr, a2, cg_of(pb, qb), src, dst, zrows, nblk)
    den = den.reshape(NC, NPAD, HEADS)
    pam, sur = _tc_final(acc, den, s8, m, b2.reshape(1, HID),
                         Wp1, bp1.reshape(1, HID // 2), Wp2, bp2.reshape(1, 4),
                         Ws1, bs1.reshape(1, HID // 2), Ws2, bs2.reshape(1, 1))
    return (pam[:n], sur[:n])


# speedup vs baseline: 16.3830x; 1.0606x over previous
import functools

import numpy as _np

import jax
import jax.numpy as jnp
from jax import lax
from jax.experimental import pallas as pl
from jax.experimental.pallas import tpu as pltpu
import jax.experimental.pallas.tpu_sc as plsc

N = 10000
DIN = 128
HID = 16
HEADS = 8
HC = HID * HEADS
NPAD = 10240
NC = 2
NS = 16
NW = NC * NS
K = 48
BLK = 1024
GRID = NPAD // BLK

_f32 = jnp.float32


def _bsplit(a):
    u = jax.lax.bitcast_convert_type(a, jnp.uint32)
    hi = jax.lax.bitcast_convert_type(u & jnp.uint32(0xFFFF0000), _f32)
    lo = a - hi
    return hi.astype(jnp.bfloat16), lo.astype(jnp.bfloat16)


def _dot2e(a, b):
    ah, al = _bsplit(a)
    bb = b.astype(jnp.bfloat16)
    return (jnp.dot(ah, bb, preferred_element_type=_f32)
            + jnp.dot(al, bb, preferred_element_type=_f32))


def _prep0_body(x_ref, wl_ref, wr_ref, sa_ref, xl_ref, xr_ref, pb_ref, qb_ref):
    h = x_ref[...]
    xl = jnp.dot(h, wl_ref[...], preferred_element_type=_f32)
    xl_ref[...] = xl
    pb_ref[...] = jnp.dot(jnp.abs(xl), sa_ref[...], preferred_element_type=_f32)
    xr = jnp.dot(h, wr_ref[...], preferred_element_type=_f32)
    xr_ref[...] = xr
    qb_ref[...] = jnp.dot(jnp.abs(xr), sa_ref[...], preferred_element_type=_f32)


def _prep_body(acc_ref, den_ref, b_ref, wl_ref, wr_ref, sa_ref, s8_ref,
               xl_ref, xr_ref, pb_ref, qb_ref):
    feat = acc_ref[0] + acc_ref[1]
    den = jnp.sum(den_ref[...], axis=0)
    denb = _dot2e(den, s8_ref[...])
    h = feat / jnp.maximum(denb, 1e-30) + b_ref[...]
    h = jnp.where(h > 0, h, jnp.exp(h) - 1.0)
    xl = jnp.dot(h, wl_ref[...], preferred_element_type=_f32)
    xl_ref[...] = xl
    pb_ref[...] = jnp.dot(jnp.abs(xl), sa_ref[...], preferred_element_type=_f32)
    xr = jnp.dot(h, wr_ref[...], preferred_element_type=_f32)
    xr_ref[...] = xr
    qb_ref[...] = jnp.dot(jnp.abs(xr), sa_ref[...], preferred_element_type=_f32)


def _final_body(acc_ref, den_ref, s8_ref, m_ref, b2_ref, wp1_ref, bp1_ref,
                wp2_ref, bp2_ref, ws1_ref, bs1_ref, ws2_ref, bs2_ref,
                pam_ref, sur_ref):
    feat = acc_ref[0] + acc_ref[1]
    den = jnp.sum(den_ref[...], axis=0)
    denb = _dot2e(den, s8_ref[...])
    hn = feat / jnp.maximum(denb, 1e-30)
    h3 = _dot2e(hn, m_ref[...]) + b2_ref[...]
    r1 = jnp.maximum(jnp.dot(h3, wp1_ref[...], preferred_element_type=_f32) + bp1_ref[...], 0.0)
    pam_ref[...] = jnp.dot(r1, wp2_ref[...], preferred_element_type=_f32) + bp2_ref[...]
    r2 = jnp.maximum(jnp.dot(h3, ws1_ref[...], preferred_element_type=_f32) + bs1_ref[...], 0.0)
    sur_ref[...] = jnp.dot(r2, ws2_ref[...], preferred_element_type=_f32) + bs2_ref[...]


def _row_spec(cols):
    return pl.BlockSpec((BLK, cols), lambda i: (i, 0))


def _full_spec(shape):
    return pl.BlockSpec(shape, lambda i: tuple(0 for _ in shape))


_ACC_SPEC = pl.BlockSpec((NC, BLK, HC), lambda i: (0, i, 0))
_DEN_SPEC = pl.BlockSpec((NC, BLK, HEADS), lambda i: (0, i, 0))
_PREP_OUT_SPECS = [_row_spec(HC), _row_spec(HC), _row_spec(HEADS), _row_spec(HEADS)]
_PREP_OUT_SHAPE = [jax.ShapeDtypeStruct((NPAD, HC), _f32),
                   jax.ShapeDtypeStruct((NPAD, HC), _f32),
                   jax.ShapeDtypeStruct((NPAD, HEADS), _f32),
                   jax.ShapeDtypeStruct((NPAD, HEADS), _f32)]


def _tc_prep0(xp, wl, wr, sa):
    return pl.pallas_call(
        _prep0_body,
        grid=(GRID,),
        in_specs=[_row_spec(DIN), _full_spec((DIN, HC)), _full_spec((DIN, HC)),
                  _full_spec((HC, HEADS))],
        out_specs=_PREP_OUT_SPECS,
        out_shape=_PREP_OUT_SHAPE,
    )(xp, wl, wr, sa)


def _tc_prep(acc, den, b2d, wl, wr, sa, s8):
    return pl.pallas_call(
        _prep_body,
        grid=(GRID,),
        in_specs=[_ACC_SPEC, _DEN_SPEC,
                  _full_spec((1, HC)), _full_spec((HC, HC)), _full_spec((HC, HC)),
                  _full_spec((HC, HEADS)), _full_spec((HEADS, HC))],
        out_specs=_PREP_OUT_SPECS,
        out_shape=_PREP_OUT_SHAPE,
    )(acc, den, b2d, wl, wr, sa, s8)


def _tc_final(acc, den, s8, m, b2, wp1, bp1, wp2, bp2, ws1, bs1, ws2, bs2):
    return pl.pallas_call(
        _final_body,
        grid=(GRID,),
        in_specs=[_ACC_SPEC, _DEN_SPEC,
                  _full_spec((HEADS, HC)), _full_spec((HC, HID)), _full_spec((1, HID)),
                  _full_spec((HID, HID // 2)), _full_spec((1, HID // 2)),
                  _full_spec((HID // 2, 4)), _full_spec((1, 4)),
                  _full_spec((HID, HID // 2)), _full_spec((1, HID // 2)),
                  _full_spec((HID // 2, 1)), _full_spec((1, 1))],
        out_specs=[_row_spec(4), _row_spec(1)],
        out_shape=[jax.ShapeDtypeStruct((NPAD, 4), _f32),
                   jax.ShapeDtypeStruct((NPAD, 1), _f32)],
    )(acc, den, s8, m, b2, wp1, bp1, wp2, bp2, ws1, bs1, ws2, bs2)


NG = NPAD // 16


def _sc_edge_body(nblk, ea, xl_hbm, xr_hbm, a_hbm, cg_hbm, src_hbm, dst_hbm, z_hbm,
                  accf_hbm, den_hbm,
                  sidx0, didx0, xlg0, xrg0,
                  sidx1, didx1, xlg1, xrg1,
                  orows, od, d16, av, cgv, semA, semB, acc, accd):
    c = lax.axis_index("c")
    s = lax.axis_index("s")
    wid = s * NC + c
    rows_per_sub = NPAD // NS
    grows_per_sub = NG // NS

    pltpu.sync_copy(z_hbm, acc.at[pl.ds(s * rows_per_sub, rows_per_sub)])
    pltpu.sync_copy(z_hbm.at[pl.ds(0, grows_per_sub)],
                    accd.at[pl.ds(s * grows_per_sub, grows_per_sub)])
    pltpu.sync_copy(a_hbm, av)
    pltpu.sync_copy(cg_hbm, cgv)
    plsc.subcore_barrier()

    lanes = lax.iota(jnp.int32, 16)
    lanes_f = lanes.astype(_f32)
    onehots = []
    for h in range(HEADS):
        dh = lanes_f - float(h)
        onehots.append(jnp.maximum(1.0 - dh * dh, 0.0))

    sets = [(sidx0, didx0, xlg0, xrg0, semA),
            (sidx1, didx1, xlg1, xrg1, semB)]

    def fetch(b, st):
        sidx, didx, xlg, xrg, sem = st
        base = wid * ea + b * K
        pltpu.sync_copy(src_hbm.at[pl.ds(base, K)], sidx)
        pltpu.sync_copy(dst_hbm.at[pl.ds(base, K)], didx)
        pltpu.async_copy(xl_hbm.at[sidx], xlg, sem)
        pltpu.async_copy(xr_hbm.at[didx], xrg, sem)

    def drain(st):
        _, _, xlg, xrg, sem = st
        pltpu.make_async_copy(xl_hbm.at[pl.ds(0, K)], xlg, sem).wait()
        pltpu.make_async_copy(xr_hbm.at[pl.ds(0, K)], xrg, sem).wait()

    def compute(st):
        _, didx, xlg, xrg, _ = st
        for j in range(K // 16):
            d16[pl.ds(j * 16, 16)] = jnp.right_shift(didx[pl.ds(j * 16, 16)], 4)

        def edge_body(e, carry2):
            pacc = jnp.zeros((16,), _f32)
            for h in range(HEADS):
                xlv = xlg[e, pl.ds(h * HID, HID)]
                xrv = xrg[e, pl.ds(h * HID, HID)]
                v = xlv + xrv
                v = jnp.maximum(v, 0.2 * v)
                t = v * av[h]
                for k in (1, 2, 4, 8):
                    t = t + t.at[lanes ^ k].get(mode="promise_in_bounds")
                pv = jnp.exp(t - cgv[h])
                orows[e, pl.ds(h * HID, HID)] = pv * xlv
                pacc = pacc + pv * onehots[h]
            eb = (e // 16) * 16
            dchunk = didx[pl.ds(eb, 16)]
            dsplat = dchunk.at[jnp.full((16,), e - eb, jnp.int32)].get(
                mode="promise_in_bounds")
            s16 = dsplat & 15
            m = (s16 & 1).astype(_f32)
            rot8 = pacc.at[lanes ^ 8].get(mode="promise_in_bounds")
            rot = pacc + m * (rot8 - pacc)
            qf = (s16 >> 1).astype(_f32)
            for j in range(8):
                dj = qf - float(j)
                mj = jnp.maximum(1.0 - dj * dj, 0.0)
                od[e, pl.ds(j * 16, 16)] = rot * mj
            return carry2

        lax.fori_loop(0, K, edge_body, 0)
        pltpu.sync_copy(orows, acc.at[didx], add=True)
        pltpu.sync_copy(od, accd.at[d16], add=True)

    fetch(0, sets[0])

    def pair_body(i, carry):
        for j in (0, 1):
            b = 2 * i + j
            fetch(b + 1, sets[1 - j])
            drain(sets[j])
            compute(sets[j])
        return carry

    lax.fori_loop(0, nblk // 2, pair_body, 0)
    drain(sets[0])
    plsc.subcore_barrier()
    pltpu.sync_copy(acc.at[pl.ds(s * rows_per_sub, rows_per_sub)],
                    accf_hbm.at[c, pl.ds(s * rows_per_sub, rows_per_sub)])
    pltpu.sync_copy(accd.at[pl.ds(s * grows_per_sub, grows_per_sub)],
                    den_hbm.at[c, pl.ds(s * grows_per_sub, grows_per_sub)])


def _sc_edge(xl, xr, a, cgb, srcm, dstm, zrows, nblk):
    ea = srcm.shape[0] // NW
    srcm = srcm
    mesh = plsc.VectorSubcoreMesh(core_axis_name="c", subcore_axis_name="s")
    f = pl.kernel(
        functools.partial(_sc_edge_body, nblk, ea),
        out_type=(jax.ShapeDtypeStruct((NC, NPAD, HC), _f32),
                  jax.ShapeDtypeStruct((NC, NG, HC), _f32)),
        mesh=mesh,
        scratch_types=(
            [pltpu.VMEM((K,), jnp.int32)] * 2
            + [pltpu.VMEM((K, HC), _f32)] * 2
            + [pltpu.VMEM((K,), jnp.int32)] * 2
            + [pltpu.VMEM((K, HC), _f32)] * 2
            + [pltpu.VMEM((K, HC), _f32)] * 2
            + [pltpu.VMEM((K,), jnp.int32)]
            + [pltpu.VMEM((HEADS, HID), _f32)] * 2
            + [pltpu.SemaphoreType.DMA, pltpu.SemaphoreType.DMA]
            + [pltpu.VMEM_SHARED((NPAD, HC), _f32),
               pltpu.VMEM_SHARED((NG, HC), _f32)]
        ),
    )
    return f(xl, xr, a, cgb, srcm, dstm, zrows)


def _blockdiag_absa(a):
    return (jnp.abs(a)[:, :, None] * jnp.eye(HEADS, dtype=_f32)[:, None, :]).reshape(HC, HEADS)


def kernel(x, edge_index, Wl0, Wr0, a0, b0, Wl1, Wr1, a1, b1, Wl2, Wr2, a2, b2,
           Wp1, bp1, Wp2, bp2, Ws1, bs1, Ws2, bs2):
    n = x.shape[0]
    e = edge_index.shape[1]
    etot = e + n
    ept = ((etot + NW * 2 * K - 1) // (NW * 2 * K)) * 2 * K
    epad = ept * NW
    nblk = ept // K

    loop = jnp.arange(n, dtype=edge_index.dtype)
    fill_src = jnp.zeros((epad - etot,), edge_index.dtype)
    fill_dst = jnp.full((epad - etot,), NPAD - 2, edge_index.dtype)
    xtr = ((ept + K + 127) // 128) * 128 - ept
    xtr_src = jnp.zeros((NW, xtr), edge_index.dtype)
    xtr_dst = jnp.full((NW, xtr), NPAD - 2, edge_index.dtype)
    src = jnp.concatenate(
        [jnp.concatenate([edge_index[0], loop, fill_src]).reshape(NW, ept), xtr_src],
        axis=1).reshape(-1)
    dst = jnp.concatenate(
        [jnp.concatenate([edge_index[1], loop, fill_dst]).reshape(NW, ept), xtr_dst],
        axis=1).reshape(-1)

    xp = jnp.pad(x, ((0, NPAD - n), (0, 0)))
    zrows = jnp.zeros((NPAD // NS, HC), _f32)
    s8 = jnp.kron(jnp.eye(HEADS, dtype=_f32), jnp.ones((1, HID), _f32))
    m = jnp.kron(jnp.ones((HEADS, 1), _f32), jnp.eye(HID, dtype=_f32)) / HEADS

    def cg_of(pb, qb):
        c = jnp.max(pb, axis=0) + jnp.max(qb, axis=0)
        return jnp.broadcast_to(c[:, None], (HEADS, HID)).astype(_f32)

    xl, xr, pb, qb = _tc_prep0(xp, Wl0, Wr0, _blockdiag_absa(a0))
    acc, den = _sc_edge(xl, xr, a0, cg_of(pb, qb), src, dst, zrows, nblk)
    den = den.reshape(NC, NPAD, HEADS)
    xl, xr, pb, qb = _tc_prep(acc, den, b0.reshape(1, HC), Wl1, Wr1,
                              _blockdiag_absa(a1), s8)
    acc, den = _sc_edge(xl, xr, a1, cg_of(pb, qb), src, dst, zrows, nblk)
    den = den.reshape(NC, NPAD, HEADS)
    xl, xr, pb, qb = _tc_prep(acc, den, b1.reshape(1, HC), Wl2, Wr2,
                              _blockdiag_absa(a2), s8)
    acc, den = _sc_edge(xl, xr, a2, cg_of(pb, qb), src, dst, zrows, nblk)
    den = den.reshape(NC, NPAD, HEADS)
    pam, sur = _tc_final(acc, den, s8, m, b2.reshape(1, HID),
                         Wp1, bp1.reshape(1, HID // 2), Wp2, bp2.reshape(1, 4),
                         Ws1, bs1.reshape(1, HID // 2), Ws2, bs2.reshape(1, 1))
    return (pam[:n], sur[:n])
